# baseline (speedup 1.0000x reference)
import functools

import jax
import jax.numpy as jnp
from jax import lax
from jax.experimental import pallas as pl
from jax.experimental.pallas import tpu as pltpu

SEQ = 8192
D = 768
BATCH = 4
TOTAL = BATCH * SEQ
BLK = 512
GRID = TOTAL // BLK


def _tc_body(idx_ref, table_ref, out_ref):
    i = pl.program_id(0)

    def inner(j, carry):
        r = idx_ref[i * BLK + j]
        out_ref[pl.ds(j, 1), :] = table_ref[pl.ds(r, 1), :]
        return carry

    lax.fori_loop(0, BLK, inner, 0, unroll=8)


@jax.jit
def kernel(t, weight):
    idx = t.reshape(TOTAL).astype(jnp.int32)
    out = pl.pallas_call(
        _tc_body,
        grid_spec=pltpu.PrefetchScalarGridSpec(
            num_scalar_prefetch=1,
            grid=(GRID,),
            in_specs=[
                pl.BlockSpec((SEQ, D), lambda i, idx_ref: (0, 0)),
            ],
            out_specs=pl.BlockSpec((BLK, D), lambda i, idx_ref: (i, 0)),
        ),
        out_shape=jax.ShapeDtypeStruct((TOTAL, D), jnp.float32),
    )(idx, weight)
    return out.reshape(BATCH, SEQ, D)
